# baseline (speedup 1.0000x reference)
import functools

import jax
import jax.numpy as jnp
from jax import lax
from jax.experimental import pallas as pl
from jax.experimental.pallas import tpu as pltpu
from jax.experimental.pallas import tpu_sc as plsc

N_VOCAB = 1000000
D = 64
B = 1024
NC, NS, L = 2, 16, 16
NW = NC * NS
BPW = B // NW

@functools.cache
def _build_sc_gather_dot():
  mesh = plsc.VectorSubcoreMesh(
      core_axis_name="c", subcore_axis_name="s", num_cores=NC, num_subcores=NS)

  @functools.partial(
      pl.kernel,
      out_type=(
          jax.ShapeDtypeStruct((B,), jnp.float32),
          jax.ShapeDtypeStruct((B,), jnp.float32),
      ),
      mesh=mesh,
      compiler_params=pltpu.CompilerParams(
          needs_layout_passes=False, use_tc_tiling_on_sc=False),
      scratch_types=[
        pltpu.VMEM((BPW,), jnp.int32),
        pltpu.VMEM((BPW,), jnp.int32),
        pltpu.VMEM((BPW, D), jnp.float32),
        pltpu.VMEM((BPW, D), jnp.float32),
        pltpu.VMEM((BPW,), jnp.float32),
        pltpu.VMEM((BPW,), jnp.float32),
        pltpu.VMEM((BPW,), jnp.float32),
        pltpu.VMEM((BPW,), jnp.float32),
          pltpu.SemaphoreType.DMA,
          pltpu.SemaphoreType.DMA,
          pltpu.SemaphoreType.DMA,
          pltpu.SemaphoreType.DMA,
      ],
  )
  def _sc_gather_dot(wi_hbm, wj_hbm, emb_hbm, bi_hbm, bj_hbm,
                     dot_hbm, bs_hbm,
                     idx_i, idx_j, ei, ej, bi, bj, dotv, bsv,
                     sem0, sem1, sem2, sem3):
    wid = lax.axis_index("s") * NC + lax.axis_index("c")
    base = wid * BPW
    pltpu.sync_copy(wi_hbm.at[pl.ds(base, BPW)], idx_i)
    pltpu.sync_copy(wj_hbm.at[pl.ds(base, BPW)], idx_j)
    cp0 = pltpu.async_copy(emb_hbm.at[idx_i], ei, sem0)
    cp1 = pltpu.async_copy(emb_hbm.at[idx_j], ej, sem1)
    cp2 = pltpu.async_copy(bi_hbm.at[idx_i], bi, sem2)
    cp3 = pltpu.async_copy(bj_hbm.at[idx_j], bj, sem3)
    cp0.wait()
    cp1.wait()
    cp2.wait()
    cp3.wait()
    iota = lax.iota(jnp.int32, L)
    for g in range(BPW // L):
      dvec = jnp.zeros((L,), jnp.float32)
      for r in range(L):
        row = g * L + r
        acc = ei[row, pl.ds(0, L)] * ej[row, pl.ds(0, L)]
        for c in range(1, D // L):
          acc = acc + ei[row, pl.ds(c * L, L)] * ej[row, pl.ds(c * L, L)]
        dvec = jnp.where(iota == r, jnp.sum(acc), dvec)
      dotv[pl.ds(g * L, L)] = dvec
      bsv[pl.ds(g * L, L)] = bi[pl.ds(g * L, L)] + bj[pl.ds(g * L, L)]
    pltpu.sync_copy(dotv, dot_hbm.at[pl.ds(base, BPW)])
    pltpu.sync_copy(bsv, bs_hbm.at[pl.ds(base, BPW)])

  return _sc_gather_dot


_ROWS = 128


def _bcast_body(dot_ref, bs_ref, out_ref):
    out_ref[...] = bs_ref[...] + dot_ref[...]


_bcast = pl.pallas_call(
    _bcast_body,
    grid=(B // _ROWS,),
    in_specs=[
        pl.BlockSpec((1, B), lambda i: (0, 0)),
        pl.BlockSpec((_ROWS, 1), lambda i: (i, 0)),
    ],
    out_specs=pl.BlockSpec((_ROWS, B), lambda i: (i, 0)),
    out_shape=jax.ShapeDtypeStruct((B, B), jnp.float32),
)


def kernel(w_i, w_j, embedding_i, embedding_j, bias_i, bias_j):
    del embedding_j
    dot, bs = _build_sc_gather_dot()(
        w_i.astype(jnp.int32), w_j.astype(jnp.int32), embedding_i,
        bias_i.reshape(N_VOCAB), bias_j.reshape(N_VOCAB))
    return _bcast(dot.reshape(1, B), bs.reshape(B, 1))


# speedup vs baseline: 2.8046x; 2.8046x over previous
import functools

import jax
import jax.numpy as jnp
from jax import lax
from jax.experimental import pallas as pl
from jax.experimental.pallas import tpu as pltpu
from jax.experimental.pallas import tpu_sc as plsc

N_VOCAB = 1000000
D = 64
B = 1024
NC, NS, L = 2, 16, 16
NW = NC * NS
BPW = B // NW


@functools.cache
def _build_sc_bias_sum():
  mesh = plsc.VectorSubcoreMesh(
      core_axis_name="c", subcore_axis_name="s", num_cores=NC, num_subcores=NS)

  @functools.partial(
      pl.kernel,
      out_type=jax.ShapeDtypeStruct((B,), jnp.float32),
      mesh=mesh,
      compiler_params=pltpu.CompilerParams(
          needs_layout_passes=False, use_tc_tiling_on_sc=False),
      scratch_types=[
          pltpu.VMEM((BPW,), jnp.int32),
          pltpu.VMEM((BPW,), jnp.int32),
          pltpu.VMEM((BPW,), jnp.float32),
          pltpu.VMEM((BPW,), jnp.float32),
          pltpu.VMEM((BPW,), jnp.float32),
          pltpu.SemaphoreType.DMA,
          pltpu.SemaphoreType.DMA,
      ],
  )
  def _sc_bias_sum(wi_hbm, wj_hbm, bi_hbm, bj_hbm, bs_hbm,
                   idx_i, idx_j, bi, bj, bsv, sem0, sem1):
    wid = lax.axis_index("s") * NC + lax.axis_index("c")
    base = wid * BPW
    pltpu.sync_copy(wi_hbm.at[pl.ds(base, BPW)], idx_i)
    pltpu.sync_copy(wj_hbm.at[pl.ds(base, BPW)], idx_j)
    cp0 = pltpu.async_copy(bi_hbm.at[idx_i], bi, sem0)
    cp1 = pltpu.async_copy(bj_hbm.at[idx_j], bj, sem1)
    cp0.wait()
    cp1.wait()
    for g in range(BPW // L):
      sl = pl.ds(g * L, L)
      bsv[sl] = bi[sl] + bj[sl]
    pltpu.sync_copy(bsv, bs_hbm.at[pl.ds(base, BPW)])

  return _sc_bias_sum


_SLOTS = 8
_STEPS = B // _SLOTS


def _gather_dot_body(wi_ref, wj_ref, *refs):
  blk_i = refs[:_SLOTS]
  blk_j = refs[_SLOTS:2 * _SLOTS]
  out_ref = refs[2 * _SLOTS]
  i = pl.program_id(0)
  lane = lax.broadcasted_iota(jnp.int32, (D, 128), 1)
  out_lane = lax.broadcasted_iota(jnp.int32, (1, B), 1)
  cur = jnp.where(i == 0, jnp.zeros((1, B), jnp.float32), out_ref[...])
  for k in range(_SLOTS):
    n = i * _SLOTS + k
    li = lax.rem(wi_ref[n], 128)
    lj = lax.rem(wj_ref[n], 128)
    a = jnp.sum(jnp.where(lane == li, blk_i[k][...], 0.0),
                axis=1, keepdims=True)
    b = jnp.sum(jnp.where(lane == lj, blk_j[k][...], 0.0),
                axis=1, keepdims=True)
    s = jnp.sum(a * b)
    cur = jnp.where(out_lane == n, s, cur)
  out_ref[...] = cur


def _make_gather_dot():
  in_specs = []
  for k in range(_SLOTS):
    in_specs.append(pl.BlockSpec(
        (D, 128),
        lambda i, wi, wj, k=k: (0, wi[i * _SLOTS + k] // 128)))
  for k in range(_SLOTS):
    in_specs.append(pl.BlockSpec(
        (D, 128),
        lambda i, wi, wj, k=k: (0, wj[i * _SLOTS + k] // 128)))
  grid_spec = pltpu.PrefetchScalarGridSpec(
      num_scalar_prefetch=2,
      grid=(_STEPS,),
      in_specs=in_specs,
      out_specs=pl.BlockSpec((1, B), lambda i, wi, wj: (0, 0)),
  )
  return pl.pallas_call(
      _gather_dot_body,
      grid_spec=grid_spec,
      out_shape=jax.ShapeDtypeStruct((1, B), jnp.float32),
  )


@functools.cache
def _gather_dot():
  return _make_gather_dot()


_ROWS = 128


def _bcast_body(dot_ref, bs_ref, out_ref):
    out_ref[...] = bs_ref[...] + dot_ref[...]


_bcast = pl.pallas_call(
    _bcast_body,
    grid=(B // _ROWS,),
    in_specs=[
        pl.BlockSpec((1, B), lambda i: (0, 0)),
        pl.BlockSpec((_ROWS, 1), lambda i: (i, 0)),
    ],
    out_specs=pl.BlockSpec((_ROWS, B), lambda i: (i, 0)),
    out_shape=jax.ShapeDtypeStruct((B, B), jnp.float32),
)


def kernel(w_i, w_j, embedding_i, embedding_j, bias_i, bias_j):
    del embedding_j
    w_i = w_i.astype(jnp.int32)
    w_j = w_j.astype(jnp.int32)
    bs = _build_sc_bias_sum()(
        w_i, w_j, bias_i.reshape(N_VOCAB), bias_j.reshape(N_VOCAB))
    emb_t = embedding_i.T
    dot = _gather_dot()(w_i, w_j, *([emb_t] * _SLOTS), *([emb_t] * _SLOTS))
    return _bcast(dot, bs.reshape(B, 1))


# speedup vs baseline: 2.9958x; 1.0682x over previous
import functools

import jax
import jax.numpy as jnp
from jax import lax
from jax.experimental import pallas as pl
from jax.experimental.pallas import tpu as pltpu
from jax.experimental.pallas import tpu_sc as plsc

N_VOCAB = 1000000
D = 64
B = 1024
NC, NS, L = 2, 16, 16
NW = NC * NS
BPW = B // NW


@functools.cache
def _build_sc_bias_sum():
  mesh = plsc.VectorSubcoreMesh(
      core_axis_name="c", subcore_axis_name="s", num_cores=NC, num_subcores=NS)

  @functools.partial(
      pl.kernel,
      out_type=jax.ShapeDtypeStruct((B,), jnp.float32),
      mesh=mesh,
      compiler_params=pltpu.CompilerParams(
          needs_layout_passes=False, use_tc_tiling_on_sc=False),
      scratch_types=[
          pltpu.VMEM((BPW,), jnp.int32),
          pltpu.VMEM((BPW,), jnp.int32),
          pltpu.VMEM((BPW,), jnp.float32),
          pltpu.VMEM((BPW,), jnp.float32),
          pltpu.VMEM((BPW,), jnp.float32),
          pltpu.SemaphoreType.DMA,
          pltpu.SemaphoreType.DMA,
      ],
  )
  def _sc_bias_sum(wi_hbm, wj_hbm, bi_hbm, bj_hbm, bs_hbm,
                   idx_i, idx_j, bi, bj, bsv, sem0, sem1):
    wid = lax.axis_index("s") * NC + lax.axis_index("c")
    base = wid * BPW
    pltpu.sync_copy(wi_hbm.at[pl.ds(base, BPW)], idx_i)
    pltpu.sync_copy(wj_hbm.at[pl.ds(base, BPW)], idx_j)
    cp0 = pltpu.async_copy(bi_hbm.at[idx_i], bi, sem0)
    cp1 = pltpu.async_copy(bj_hbm.at[idx_j], bj, sem1)
    cp0.wait()
    cp1.wait()
    for g in range(BPW // L):
      sl = pl.ds(g * L, L)
      bsv[sl] = bi[sl] + bj[sl]
    pltpu.sync_copy(bsv, bs_hbm.at[pl.ds(base, BPW)])

  return _sc_bias_sum


_SLOTS = 16
_STEPS = B // _SLOTS


def _gather_dot_body(wi_ref, wj_ref, wiv_ref, wjv_ref, *refs):
  blk_i = refs[:_SLOTS]
  blk_j = refs[_SLOTS:2 * _SLOTS]
  out_ref = refs[2 * _SLOTS]
  i = pl.program_id(0)
  w = _SLOTS * 128
  n_iota = lax.broadcasted_iota(jnp.int32, (_SLOTS, B), 1)
  k_iota = lax.broadcasted_iota(jnp.int32, (_SLOTS, B), 0)
  sel = jnp.where(n_iota == i * _SLOTS + k_iota, 1.0, 0.0)
  cdn = (((1,), (1,)), ((), ()))
  wiv = lax.dot_general(sel, wiv_ref[...].astype(jnp.float32), cdn)
  wjv = lax.dot_general(sel, wjv_ref[...].astype(jnp.float32), cdn)
  li = wiv.astype(jnp.int32) % 128
  lj = wjv.astype(jnp.int32) % 128
  c_iota = lax.broadcasted_iota(jnp.int32, (_SLOTS, w), 1)
  r_iota = lax.broadcasted_iota(jnp.int32, (_SLOTS, w), 0)
  blk_of_c = c_iota // 128
  lane_of_c = c_iota % 128
  oh_i = jnp.where((blk_of_c == r_iota) & (lane_of_c == li), 1.0, 0.0)
  oh_j = jnp.where((blk_of_c == r_iota) & (lane_of_c == lj), 1.0, 0.0)
  cat_i = jnp.concatenate([r[...] for r in blk_i], axis=1)
  cat_j = jnp.concatenate([r[...] for r in blk_j], axis=1)
  a = lax.dot_general(cat_i, oh_i, cdn)
  b = lax.dot_general(cat_j, oh_j, cdn)
  dots = jnp.sum(a * b, axis=0, keepdims=True)
  contrib = lax.dot_general(dots, sel, (((1,), (0,)), ((), ())),
                            precision=lax.Precision.HIGHEST)
  prev = jnp.where(i == 0, jnp.zeros((1, B), jnp.float32), out_ref[...])
  out_ref[...] = prev + contrib


def _make_gather_dot():
  in_specs = [
      pl.BlockSpec((1, B), lambda i, wi, wj: (0, 0)),
      pl.BlockSpec((1, B), lambda i, wi, wj: (0, 0)),
  ]
  for k in range(_SLOTS):
    in_specs.append(pl.BlockSpec(
        (D, 128),
        lambda i, wi, wj, k=k: (0, wi[i * _SLOTS + k] // 128)))
  for k in range(_SLOTS):
    in_specs.append(pl.BlockSpec(
        (D, 128),
        lambda i, wi, wj, k=k: (0, wj[i * _SLOTS + k] // 128)))
  grid_spec = pltpu.PrefetchScalarGridSpec(
      num_scalar_prefetch=2,
      grid=(_STEPS,),
      in_specs=in_specs,
      out_specs=pl.BlockSpec((1, B), lambda i, wi, wj: (0, 0)),
  )
  return pl.pallas_call(
      _gather_dot_body,
      grid_spec=grid_spec,
      out_shape=jax.ShapeDtypeStruct((1, B), jnp.float32),
  )


@functools.cache
def _gather_dot():
  return _make_gather_dot()


_ROWS = 128


def _bcast_body(dot_ref, bs_ref, out_ref):
    out_ref[...] = bs_ref[...] + dot_ref[...]


_bcast = pl.pallas_call(
    _bcast_body,
    grid=(B // _ROWS,),
    in_specs=[
        pl.BlockSpec((1, B), lambda i: (0, 0)),
        pl.BlockSpec((_ROWS, 1), lambda i: (i, 0)),
    ],
    out_specs=pl.BlockSpec((_ROWS, B), lambda i: (i, 0)),
    out_shape=jax.ShapeDtypeStruct((B, B), jnp.float32),
)


def kernel(w_i, w_j, embedding_i, embedding_j, bias_i, bias_j):
    del embedding_j
    w_i = w_i.astype(jnp.int32)
    w_j = w_j.astype(jnp.int32)
    bs = _build_sc_bias_sum()(
        w_i, w_j, bias_i.reshape(N_VOCAB), bias_j.reshape(N_VOCAB))
    emb_t = embedding_i.T
    dot = _gather_dot()(w_i, w_j,
                        (w_i % 128).reshape(1, B), (w_j % 128).reshape(1, B),
                        *([emb_t] * _SLOTS), *([emb_t] * _SLOTS))
    return _bcast(dot, bs.reshape(B, 1))


# speedup vs baseline: 3.1295x; 1.0446x over previous
import functools

import jax
import jax.numpy as jnp
from jax import lax
from jax.experimental import pallas as pl
from jax.experimental.pallas import tpu as pltpu
from jax.experimental.pallas import tpu_sc as plsc

N_VOCAB = 1000000
D = 64
B = 1024
NC, NS, L = 2, 16, 16
NW = NC * NS
BPW = B // NW


@functools.cache
def _build_sc_bias_sum():
  mesh = plsc.VectorSubcoreMesh(
      core_axis_name="c", subcore_axis_name="s", num_cores=NC, num_subcores=NS)

  @functools.partial(
      pl.kernel,
      out_type=jax.ShapeDtypeStruct((B,), jnp.float32),
      mesh=mesh,
      compiler_params=pltpu.CompilerParams(
          needs_layout_passes=False, use_tc_tiling_on_sc=False),
      scratch_types=[
          pltpu.VMEM((BPW,), jnp.int32),
          pltpu.VMEM((BPW,), jnp.int32),
          pltpu.VMEM((BPW,), jnp.float32),
          pltpu.VMEM((BPW,), jnp.float32),
          pltpu.VMEM((BPW,), jnp.float32),
          pltpu.SemaphoreType.DMA,
          pltpu.SemaphoreType.DMA,
      ],
  )
  def _sc_bias_sum(wi_hbm, wj_hbm, bi_hbm, bj_hbm, bs_hbm,
                   idx_i, idx_j, bi, bj, bsv, sem0, sem1):
    wid = lax.axis_index("s") * NC + lax.axis_index("c")
    base = wid * BPW
    pltpu.sync_copy(wi_hbm.at[pl.ds(base, BPW)], idx_i)
    pltpu.sync_copy(wj_hbm.at[pl.ds(base, BPW)], idx_j)
    cp0 = pltpu.async_copy(bi_hbm.at[idx_i], bi, sem0)
    cp1 = pltpu.async_copy(bj_hbm.at[idx_j], bj, sem1)
    cp0.wait()
    cp1.wait()
    for g in range(BPW // L):
      sl = pl.ds(g * L, L)
      bsv[sl] = bi[sl] + bj[sl]
    pltpu.sync_copy(bsv, bs_hbm.at[pl.ds(base, BPW)])

  return _sc_bias_sum


_SLOTS = 32
_STEPS = B // _SLOTS


def _gather_dot_body(wi_ref, wj_ref, wiv_ref, wjv_ref, *refs):
  blk_i = refs[:_SLOTS]
  blk_j = refs[_SLOTS:2 * _SLOTS]
  out_ref = refs[2 * _SLOTS]
  i = pl.program_id(0)
  w = _SLOTS * 128
  n_iota = lax.broadcasted_iota(jnp.int32, (_SLOTS, B), 1)
  k_iota = lax.broadcasted_iota(jnp.int32, (_SLOTS, B), 0)
  sel = jnp.where(n_iota == i * _SLOTS + k_iota, 1.0, 0.0)
  cdn = (((1,), (1,)), ((), ()))
  wiv = lax.dot_general(sel, wiv_ref[...].astype(jnp.float32), cdn)
  wjv = lax.dot_general(sel, wjv_ref[...].astype(jnp.float32), cdn)
  li = wiv.astype(jnp.int32) % 128
  lj = wjv.astype(jnp.int32) % 128
  c_iota = lax.broadcasted_iota(jnp.int32, (_SLOTS, w), 1)
  r_iota = lax.broadcasted_iota(jnp.int32, (_SLOTS, w), 0)
  blk_of_c = c_iota // 128
  lane_of_c = c_iota % 128
  oh_i = jnp.where((blk_of_c == r_iota) & (lane_of_c == li), 1.0, 0.0)
  oh_j = jnp.where((blk_of_c == r_iota) & (lane_of_c == lj), 1.0, 0.0)
  cat_i = jnp.concatenate([r[...] for r in blk_i], axis=1)
  cat_j = jnp.concatenate([r[...] for r in blk_j], axis=1)
  a = lax.dot_general(cat_i, oh_i, cdn)
  b = lax.dot_general(cat_j, oh_j, cdn)
  dots = jnp.sum(a * b, axis=0, keepdims=True)
  contrib = lax.dot_general(dots, sel, (((1,), (0,)), ((), ())),
                            precision=lax.Precision.HIGHEST)
  prev = jnp.where(i == 0, jnp.zeros((1, B), jnp.float32), out_ref[...])
  out_ref[...] = prev + contrib


def _make_gather_dot():
  in_specs = [
      pl.BlockSpec((1, B), lambda i, wi, wj: (0, 0)),
      pl.BlockSpec((1, B), lambda i, wi, wj: (0, 0)),
  ]
  for k in range(_SLOTS):
    in_specs.append(pl.BlockSpec(
        (D, 128),
        lambda i, wi, wj, k=k: (0, wi[i * _SLOTS + k] // 128)))
  for k in range(_SLOTS):
    in_specs.append(pl.BlockSpec(
        (D, 128),
        lambda i, wi, wj, k=k: (0, wj[i * _SLOTS + k] // 128)))
  grid_spec = pltpu.PrefetchScalarGridSpec(
      num_scalar_prefetch=2,
      grid=(_STEPS,),
      in_specs=in_specs,
      out_specs=pl.BlockSpec((1, B), lambda i, wi, wj: (0, 0)),
  )
  return pl.pallas_call(
      _gather_dot_body,
      grid_spec=grid_spec,
      out_shape=jax.ShapeDtypeStruct((1, B), jnp.float32),
  )


@functools.cache
def _gather_dot():
  return _make_gather_dot()


_ROWS = 128


def _bcast_body(dot_ref, bs_ref, out_ref):
    out_ref[...] = bs_ref[...] + dot_ref[...]


_bcast = pl.pallas_call(
    _bcast_body,
    grid=(B // _ROWS,),
    in_specs=[
        pl.BlockSpec((1, B), lambda i: (0, 0)),
        pl.BlockSpec((_ROWS, 1), lambda i: (i, 0)),
    ],
    out_specs=pl.BlockSpec((_ROWS, B), lambda i: (i, 0)),
    out_shape=jax.ShapeDtypeStruct((B, B), jnp.float32),
)


def kernel(w_i, w_j, embedding_i, embedding_j, bias_i, bias_j):
    del embedding_j
    w_i = w_i.astype(jnp.int32)
    w_j = w_j.astype(jnp.int32)
    bs = _build_sc_bias_sum()(
        w_i, w_j, bias_i.reshape(N_VOCAB), bias_j.reshape(N_VOCAB))
    emb_t = embedding_i.T
    dot = _gather_dot()(w_i, w_j,
                        (w_i % 128).reshape(1, B), (w_j % 128).reshape(1, B),
                        *([emb_t] * _SLOTS), *([emb_t] * _SLOTS))
    return _bcast(dot, bs.reshape(B, 1))


# speedup vs baseline: 3.1711x; 1.0133x over previous
import functools

import jax
import jax.numpy as jnp
from jax import lax
from jax.experimental import pallas as pl
from jax.experimental.pallas import tpu as pltpu
from jax.experimental.pallas import tpu_sc as plsc

N_VOCAB = 1000000
D = 64
B = 1024
NC, NS, L = 2, 16, 16
NW = NC * NS
BPW = B // NW


@functools.cache
def _build_sc_bias_sum():
  mesh = plsc.VectorSubcoreMesh(
      core_axis_name="c", subcore_axis_name="s", num_cores=NC, num_subcores=NS)

  @functools.partial(
      pl.kernel,
      out_type=jax.ShapeDtypeStruct((B,), jnp.float32),
      mesh=mesh,
      compiler_params=pltpu.CompilerParams(
          needs_layout_passes=False, use_tc_tiling_on_sc=False),
      scratch_types=[
          pltpu.VMEM((BPW,), jnp.int32),
          pltpu.VMEM((BPW,), jnp.int32),
          pltpu.VMEM((BPW,), jnp.float32),
          pltpu.VMEM((BPW,), jnp.float32),
          pltpu.VMEM((BPW,), jnp.float32),
          pltpu.SemaphoreType.DMA,
          pltpu.SemaphoreType.DMA,
      ],
  )
  def _sc_bias_sum(wi_hbm, wj_hbm, bi_hbm, bj_hbm, bs_hbm,
                   idx_i, idx_j, bi, bj, bsv, sem0, sem1):
    wid = lax.axis_index("s") * NC + lax.axis_index("c")
    base = wid * BPW
    pltpu.sync_copy(wi_hbm.at[pl.ds(base, BPW)], idx_i)
    pltpu.sync_copy(wj_hbm.at[pl.ds(base, BPW)], idx_j)
    cp0 = pltpu.async_copy(bi_hbm.at[idx_i], bi, sem0)
    cp1 = pltpu.async_copy(bj_hbm.at[idx_j], bj, sem1)
    cp0.wait()
    cp1.wait()
    for g in range(BPW // L):
      sl = pl.ds(g * L, L)
      bsv[sl] = bi[sl] + bj[sl]
    pltpu.sync_copy(bsv, bs_hbm.at[pl.ds(base, BPW)])

  return _sc_bias_sum


_SLOTS = 64
_STEPS = B // _SLOTS


def _gather_dot_body(wi_ref, wj_ref, wiv_ref, wjv_ref, *refs):
  blk_i = refs[:_SLOTS]
  blk_j = refs[_SLOTS:2 * _SLOTS]
  out_ref = refs[2 * _SLOTS]
  i = pl.program_id(0)
  w = _SLOTS * 128
  n_iota = lax.broadcasted_iota(jnp.int32, (_SLOTS, B), 1)
  k_iota = lax.broadcasted_iota(jnp.int32, (_SLOTS, B), 0)
  sel = jnp.where(n_iota == i * _SLOTS + k_iota, 1.0, 0.0)
  cdn = (((1,), (1,)), ((), ()))
  wiv = lax.dot_general(sel, wiv_ref[...].astype(jnp.float32), cdn)
  wjv = lax.dot_general(sel, wjv_ref[...].astype(jnp.float32), cdn)
  li = wiv.astype(jnp.int32) % 128
  lj = wjv.astype(jnp.int32) % 128
  c_iota = lax.broadcasted_iota(jnp.int32, (_SLOTS, w), 1)
  r_iota = lax.broadcasted_iota(jnp.int32, (_SLOTS, w), 0)
  blk_of_c = c_iota // 128
  lane_of_c = c_iota % 128
  oh_i = jnp.where((blk_of_c == r_iota) & (lane_of_c == li), 1.0, 0.0)
  oh_j = jnp.where((blk_of_c == r_iota) & (lane_of_c == lj), 1.0, 0.0)
  cat_i = jnp.concatenate([r[...] for r in blk_i], axis=1)
  cat_j = jnp.concatenate([r[...] for r in blk_j], axis=1)
  a = lax.dot_general(cat_i, oh_i, cdn)
  b = lax.dot_general(cat_j, oh_j, cdn)
  dots = jnp.sum(a * b, axis=0, keepdims=True)
  contrib = lax.dot_general(dots, sel, (((1,), (0,)), ((), ())),
                            precision=lax.Precision.HIGHEST)
  prev = jnp.where(i == 0, jnp.zeros((1, B), jnp.float32), out_ref[...])
  out_ref[...] = prev + contrib


def _make_gather_dot():
  in_specs = [
      pl.BlockSpec((1, B), lambda i, wi, wj: (0, 0)),
      pl.BlockSpec((1, B), lambda i, wi, wj: (0, 0)),
  ]
  for k in range(_SLOTS):
    in_specs.append(pl.BlockSpec(
        (D, 128),
        lambda i, wi, wj, k=k: (0, wi[i * _SLOTS + k] // 128)))
  for k in range(_SLOTS):
    in_specs.append(pl.BlockSpec(
        (D, 128),
        lambda i, wi, wj, k=k: (0, wj[i * _SLOTS + k] // 128)))
  grid_spec = pltpu.PrefetchScalarGridSpec(
      num_scalar_prefetch=2,
      grid=(_STEPS,),
      in_specs=in_specs,
      out_specs=pl.BlockSpec((1, B), lambda i, wi, wj: (0, 0)),
  )
  return pl.pallas_call(
      _gather_dot_body,
      grid_spec=grid_spec,
      out_shape=jax.ShapeDtypeStruct((1, B), jnp.float32),
  )


@functools.cache
def _gather_dot():
  return _make_gather_dot()


_ROWS = 128


def _bcast_body(dot_ref, bs_ref, out_ref):
    out_ref[...] = bs_ref[...] + dot_ref[...]


_bcast = pl.pallas_call(
    _bcast_body,
    grid=(B // _ROWS,),
    in_specs=[
        pl.BlockSpec((1, B), lambda i: (0, 0)),
        pl.BlockSpec((_ROWS, 1), lambda i: (i, 0)),
    ],
    out_specs=pl.BlockSpec((_ROWS, B), lambda i: (i, 0)),
    out_shape=jax.ShapeDtypeStruct((B, B), jnp.float32),
)


def kernel(w_i, w_j, embedding_i, embedding_j, bias_i, bias_j):
    del embedding_j
    w_i = w_i.astype(jnp.int32)
    w_j = w_j.astype(jnp.int32)
    bs = _build_sc_bias_sum()(
        w_i, w_j, bias_i.reshape(N_VOCAB), bias_j.reshape(N_VOCAB))
    emb_t = embedding_i.T
    dot = _gather_dot()(w_i, w_j,
                        (w_i % 128).reshape(1, B), (w_j % 128).reshape(1, B),
                        *([emb_t] * _SLOTS), *([emb_t] * _SLOTS))
    return _bcast(dot, bs.reshape(B, 1))
